# baseline (speedup 1.0000x reference)
import jax
import jax.numpy as jnp
from jax.experimental import pallas as pl
from jax.experimental.pallas import tpu as pltpu


def _split3(a):
    mask = jnp.uint32(0xFFFF0000)

    def trunc(v):
        vb = jax.lax.bitcast_convert_type(v, jnp.uint32)
        return jax.lax.bitcast_convert_type(vb & mask, jnp.float32)

    hi = trunc(a)
    r = a - hi
    mid = trunc(r)
    r2 = r - mid
    lo = trunc(r2)
    return (hi.astype(jnp.bfloat16), mid.astype(jnp.bfloat16),
            lo.astype(jnp.bfloat16))


def _chamfer_kernel(ab_ref, out_ref, *, inv_count):
    b = pl.program_id(0)
    n = ab_ref.shape[1] // 2
    acat = ab_ref[0, :n]
    bcat = ab_ref[0, n:]
    d2 = jax.lax.dot_general(
        acat, bcat, (((1,), (1,)), ((), ())),
        preferred_element_type=jnp.float32,
    )
    min_x = jnp.min(d2, axis=1)
    min_y = jnp.min(d2, axis=0)
    min_x = jnp.maximum(min_x, 0.0)
    min_y = jnp.maximum(min_y, 0.0)
    partial = jnp.sum(jnp.sqrt(1e-6 + min_x)) + jnp.sum(jnp.sqrt(1e-6 + min_y))

    @pl.when(b == 0)
    def _():
        out_ref[0, 0] = 0.0

    out_ref[0, 0] += partial * inv_count


def kernel(x, y):
    bs, n, _ = x.shape
    xsq = jnp.sum(x * x, axis=2, keepdims=True)
    ysq = jnp.sum(y * y, axis=2, keepdims=True)
    ones = jnp.ones_like(xsq)
    xa = jnp.concatenate([x, ones, xsq], axis=2)
    ya = jnp.concatenate([-2.0 * y, ysq, ones], axis=2)
    xh, xm, xl = _split3(xa)
    yh, ym, yl = _split3(ya)
    abcat = jnp.concatenate(
        [jnp.concatenate([xh, xh, xm, xh, xl, xm], axis=2),
         jnp.concatenate([yh, ym, yh, yl, yh, ym], axis=2)], axis=1)

    k = abcat.shape[2]
    import functools as _ft
    total = pl.pallas_call(
        _ft.partial(_chamfer_kernel, inv_count=1.0 / (bs * n)),
        grid=(bs,),
        in_specs=[
            pl.BlockSpec((1, 2 * n, k), lambda b: (b, 0, 0)),
        ],
        out_specs=pl.BlockSpec(memory_space=pltpu.SMEM),
        out_shape=jax.ShapeDtypeStruct((1, 1), jnp.float32),
    )(abcat)
    return total[0, 0]


# speedup vs baseline: 1.1078x; 1.1078x over previous
import jax
import jax.numpy as jnp
from jax.experimental import pallas as pl
from jax.experimental.pallas import tpu as pltpu


def _split2(a):
    mask = jnp.uint32(0xFFFF0000)

    def trunc(v):
        vb = jax.lax.bitcast_convert_type(v, jnp.uint32)
        return jax.lax.bitcast_convert_type(vb & mask, jnp.float32)

    hi = trunc(a)
    r = a - hi
    mid = trunc(r)
    return hi.astype(jnp.bfloat16), mid.astype(jnp.bfloat16)


def _chamfer_kernel(a_ref, b_ref, out_ref):
    acat = a_ref[0]
    bcat = b_ref[0]
    d2 = jax.lax.dot_general(
        acat, bcat, (((1,), (1,)), ((), ())),
        preferred_element_type=jnp.float32,
    )
    min_x = jnp.min(d2, axis=1)
    min_y = jnp.min(d2, axis=0)
    min_x = jnp.maximum(min_x, 0.0)
    min_y = jnp.maximum(min_y, 0.0)
    partial = jnp.sum(jnp.sqrt(1e-6 + min_x)) + jnp.sum(jnp.sqrt(1e-6 + min_y))
    out_ref[0, 0, 0] = partial


def kernel(x, y):
    bs, n, _ = x.shape
    xsq = jnp.sum(x * x, axis=2, keepdims=True)
    ysq = jnp.sum(y * y, axis=2, keepdims=True)
    ones = jnp.ones_like(xsq)
    xa = jnp.concatenate([x, ones, xsq], axis=2)
    ya = jnp.concatenate([-2.0 * y, ysq, ones], axis=2)
    xh, xm = _split2(xa)
    yh, ym = _split2(ya)
    acat = jnp.concatenate([xh, xh, xm, xm], axis=2)
    bcat = jnp.concatenate([yh, ym, yh, ym], axis=2)

    k = acat.shape[2]
    partials = pl.pallas_call(
        _chamfer_kernel,
        grid=(bs,),
        in_specs=[
            pl.BlockSpec((1, n, k), lambda b: (b, 0, 0)),
            pl.BlockSpec((1, n, k), lambda b: (b, 0, 0)),
        ],
        out_specs=pl.BlockSpec((1, 1, 1), lambda b: (b, 0, 0),
                               memory_space=pltpu.SMEM),
        out_shape=jax.ShapeDtypeStruct((bs, 1, 1), jnp.float32),
    )(acat, bcat)
    return jnp.sum(partials) / (bs * n)


# speedup vs baseline: 1.1725x; 1.0585x over previous
import jax
import jax.numpy as jnp
from jax.experimental import pallas as pl
from jax.experimental.pallas import tpu as pltpu


def _split2(a):
    mask = jnp.uint32(0xFFFF0000)

    def trunc(v):
        vb = jax.lax.bitcast_convert_type(v, jnp.uint32)
        return jax.lax.bitcast_convert_type(vb & mask, jnp.float32)

    hi = trunc(a)
    r = a - hi
    mid = trunc(r)
    return hi.astype(jnp.bfloat16), mid.astype(jnp.bfloat16)


def _chamfer_kernel(a_ref, b_ref, out_ref):
    acat = a_ref[0]
    bcat = b_ref[0]
    d2 = jax.lax.dot_general(
        acat, bcat, (((1,), (1,)), ((), ())),
        preferred_element_type=jnp.float32,
    )
    min_x = jnp.min(d2, axis=1)
    min_y = jnp.min(d2, axis=0)
    min_x = jnp.maximum(min_x, 0.0)
    min_y = jnp.maximum(min_y, 0.0)
    partial = jnp.sum(jnp.sqrt(1e-6 + min_x)) + jnp.sum(jnp.sqrt(1e-6 + min_y))

    @pl.when(pl.program_id(0) == 0)
    def _():
        out_ref[0, 0] = 0.0

    out_ref[0, 0] += partial


def kernel(x, y):
    bs, n, _ = x.shape
    xsq = jnp.sum(x * x, axis=2, keepdims=True)
    ysq = jnp.sum(y * y, axis=2, keepdims=True)
    ones = jnp.ones_like(xsq)
    xa = jnp.concatenate([x, ones, xsq], axis=2)
    ya = jnp.concatenate([-2.0 * y, ysq, ones], axis=2)
    xh, xm = _split2(xa)
    yh, ym = _split2(ya)
    acat = jnp.concatenate([xh, xh, xm, xm], axis=2)
    bcat = jnp.concatenate([yh, ym, yh, ym], axis=2)

    k = acat.shape[2]
    total = pl.pallas_call(
        _chamfer_kernel,
        grid=(bs,),
        in_specs=[
            pl.BlockSpec((1, n, k), lambda b: (b, 0, 0)),
            pl.BlockSpec((1, n, k), lambda b: (b, 0, 0)),
        ],
        out_specs=pl.BlockSpec(memory_space=pltpu.SMEM),
        out_shape=jax.ShapeDtypeStruct((1, 1), jnp.float32),
    )(acat, bcat)
    return total[0, 0] / (bs * n)
